# speedup vs baseline: 5.5304x; 5.5304x over previous
import functools

import jax
import jax.numpy as jnp
from jax import lax
from jax.experimental import pallas as pl
from jax.experimental.pallas import tpu as pltpu
from jax.experimental.pallas import tpu_sc as plsc

_V = 32768
_D = 256
_NC = 2
_NS = 16
_NW = _NC * _NS
_C = 128


def _make_gather(b_total):
    b_per_w = b_total // _NW
    nchunk = b_per_w // _C
    mesh = plsc.VectorSubcoreMesh(core_axis_name="c", subcore_axis_name="s")

    @functools.partial(
        pl.kernel,
        out_type=jax.ShapeDtypeStruct((b_total, _D), jnp.float32),
        mesh=mesh,
        scratch_types=[
            pltpu.VMEM((nchunk, _C), jnp.int32),
            pltpu.VMEM((_C, _D), jnp.float32),
            pltpu.SemaphoreType.DMA,
        ],
    )
    def gather_kernel(table_hbm, idx_hbm, out_hbm, idx_v, buf, sem):
        wid = lax.axis_index("s") * _NC + lax.axis_index("c")
        base = wid * b_per_w
        pltpu.sync_copy(idx_hbm.at[pl.ds(wid * nchunk, nchunk)], idx_v)

        def body(j, carry):
            pltpu.async_copy(table_hbm.at[idx_v.at[j]], buf, sem).wait()
            pltpu.sync_copy(buf, out_hbm.at[pl.ds(base + j * _C, _C)])
            return carry

        lax.fori_loop(0, nchunk, body, 0)

    return gather_kernel


def kernel(freqs_cis, seqlen, tok_idx):
    if tok_idx is None:
        return freqs_cis[0:seqlen]
    table = freqs_cis.reshape(_V, _D)
    b, s = tok_idx.shape
    b_total = b * s
    idx = tok_idx.reshape(b_total // _C, _C).astype(jnp.int32)
    out = _make_gather(b_total)(table, idx)
    return out.reshape(b, s, _D // 4, 2, 2)


# speedup vs baseline: 5.7658x; 1.0425x over previous
import functools

import jax
import jax.numpy as jnp
from jax import lax
from jax.experimental import pallas as pl
from jax.experimental.pallas import tpu as pltpu
from jax.experimental.pallas import tpu_sc as plsc

_V = 32768
_D = 256
_NC = 2
_NS = 16
_NW = _NC * _NS
_C = 128


def _make_gather(b_total):
    b_per_w = b_total // _NW
    nchunk = b_per_w // _C
    mesh = plsc.VectorSubcoreMesh(core_axis_name="c", subcore_axis_name="s")

    @functools.partial(
        pl.kernel,
        out_type=jax.ShapeDtypeStruct((b_total, _D), jnp.float32),
        mesh=mesh,
        scratch_types=[
            pltpu.VMEM((nchunk, _C), jnp.int32),
            pltpu.VMEM((_C, _D), jnp.float32),
            pltpu.VMEM((_C, _D), jnp.float32),
            pltpu.SemaphoreType.DMA,
            pltpu.SemaphoreType.DMA,
            pltpu.SemaphoreType.DMA,
            pltpu.SemaphoreType.DMA,
        ],
    )
    def gather_kernel(table_hbm, idx_hbm, out_hbm, idx_v,
                      buf0, buf1, gsem0, gsem1, ssem0, ssem1):
        wid = lax.axis_index("s") * _NC + lax.axis_index("c")
        base = wid * b_per_w
        pltpu.sync_copy(idx_hbm.at[pl.ds(wid * nchunk, nchunk)], idx_v)
        bufs = (buf0, buf1)
        gsems = (gsem0, gsem1)
        ssems = (ssem0, ssem1)

        def gather_start(t, b):
            pltpu.async_copy(table_hbm.at[idx_v.at[t]], bufs[b], gsems[b])

        def gather_wait(b):
            pltpu.make_async_copy(table_hbm.at[idx_v.at[0]], bufs[b],
                                  gsems[b]).wait()

        def scatter_start(t, b):
            pltpu.async_copy(bufs[b], out_hbm.at[pl.ds(base + t * _C, _C)],
                             ssems[b])

        def scatter_wait(b):
            pltpu.make_async_copy(bufs[b], out_hbm.at[pl.ds(base, _C)],
                                  ssems[b]).wait()

        gather_start(0, 0)

        def body(g, carry):
            for b in (0, 1):
                t = 2 * g + b
                bo = 1 - b

                @pl.when(t + 1 < nchunk)
                def _():
                    @pl.when(t >= 1)
                    def _():
                        scatter_wait(bo)

                    gather_start(t + 1, bo)

                gather_wait(b)
                scatter_start(t, b)
            return carry

        lax.fori_loop(0, nchunk // 2, body, 0)
        scatter_wait(0)
        scatter_wait(1)

    return gather_kernel


def kernel(freqs_cis, seqlen, tok_idx):
    if tok_idx is None:
        return freqs_cis[0:seqlen]
    table = freqs_cis.reshape(_V, _D)
    b, s = tok_idx.shape
    b_total = b * s
    idx = tok_idx.reshape(b_total // _C, _C).astype(jnp.int32)
    out = _make_gather(b_total)(table, idx)
    return out.reshape(b, s, _D // 4, 2, 2)


# speedup vs baseline: 8.5349x; 1.4803x over previous
import functools

import jax
import jax.numpy as jnp
from jax import lax
from jax.experimental import pallas as pl
from jax.experimental.pallas import tpu as pltpu
from jax.experimental.pallas import tpu_sc as plsc

_NC = 2
_NS = 16
_NW = _NC * _NS
_C = 128


def _make_gather(v, d, b, s):
    b_total = b * s
    b_per_w = b_total // _NW
    nchunk = b_per_w // _C
    mesh = plsc.VectorSubcoreMesh(core_axis_name="c", subcore_axis_name="s")

    @functools.partial(
        pl.kernel,
        out_type=jax.ShapeDtypeStruct((_NW, b_per_w, d), jnp.float32),
        mesh=mesh,
        scratch_types=[
            pltpu.VMEM((b_per_w,), jnp.int32),
            pltpu.VMEM((_C, d), jnp.float32),
            pltpu.VMEM((_C, d), jnp.float32),
            pltpu.SemaphoreType.DMA,
            pltpu.SemaphoreType.DMA,
            pltpu.SemaphoreType.DMA,
            pltpu.SemaphoreType.DMA,
        ],
    )
    def gather_kernel(table_hbm, idx_hbm, out_hbm, idx_v,
                      buf0, buf1, gsem0, gsem1, ssem0, ssem1):
        wid = lax.axis_index("s") * _NC + lax.axis_index("c")
        pltpu.sync_copy(idx_hbm.at[wid], idx_v)
        bufs = (buf0, buf1)
        gsems = (gsem0, gsem1)
        ssems = (ssem0, ssem1)

        def gather_start(t, bf):
            pltpu.async_copy(table_hbm.at[idx_v.at[pl.ds(t * _C, _C)]],
                             bufs[bf], gsems[bf])

        def gather_wait(bf):
            pltpu.make_async_copy(table_hbm.at[idx_v.at[pl.ds(0, _C)]],
                                  bufs[bf], gsems[bf]).wait()

        def scatter_start(t, bf):
            pltpu.async_copy(bufs[bf], out_hbm.at[wid, pl.ds(t * _C, _C)],
                             ssems[bf])

        def scatter_wait(bf):
            pltpu.make_async_copy(bufs[bf], out_hbm.at[wid, pl.ds(0, _C)],
                                  ssems[bf]).wait()

        gather_start(0, 0)

        def body(g, carry):
            for bf in (0, 1):
                t = 2 * g + bf
                bo = 1 - bf

                @pl.when(t + 1 < nchunk)
                def _():
                    @pl.when(t >= 1)
                    def _():
                        scatter_wait(bo)

                    gather_start(t + 1, bo)

                gather_wait(bf)
                scatter_start(t, bf)
            return carry

        lax.fori_loop(0, nchunk // 2, body, 0)
        scatter_wait(0)
        scatter_wait(1)

    return gather_kernel


def kernel(freqs_cis, seqlen, tok_idx):
    if tok_idx is None:
        return freqs_cis[0:seqlen]
    b, s = tok_idx.shape
    v = freqs_cis.shape[0]
    row_shape = freqs_cis.shape[1:]
    d = 1
    for n in row_shape:
        d *= n
    table = freqs_cis.reshape(v, d)
    out = _make_gather(v, d, b, s)(table, tok_idx)
    return out.reshape((b, s) + row_shape)
